# baseline (speedup 1.0000x reference)
import functools

import jax
import jax.numpy as jnp
from jax import lax
from jax.experimental import pallas as pl
from jax.experimental.pallas import tpu as pltpu
from jax.experimental.pallas import tpu_sc as plsc

N = 10000
D = 128
NZ = 16
NC = 16
NE = 160000
SIGMA = 0.5

BATCH = 128
NE_PAD = 163840
NB_TOTAL = NE_PAD // BATCH
ACC_ROWS = 10240
ZERO_ROWS = ACC_ROWS // 16
OUT_PT = 624
OUT_TAIL = N - 16 * OUT_PT

BN = 1000
GRID_R = N // BN


@functools.lru_cache(maxsize=None)
def _make_spmm(wc: int, feature_split: bool, batch: int = BATCH,
               nslots: int = 2, n_chunks: int = 5):
    nbt = NE_PAD // batch
    nb = (nbt // 16) if feature_split else (nbt // 32)
    ch_b = nb // n_chunks
    assert ch_b % 8 == 0 and ch_b % nslots == 0, (ch_b, nslots)
    mesh = plsc.VectorSubcoreMesh(core_axis_name="c", subcore_axis_name="s")
    out_sds = jax.ShapeDtypeStruct((N, wc), jnp.float32)

    @functools.partial(
        pl.kernel,
        out_type=(out_sds, out_sds),
        mesh=mesh,
        compiler_params=pltpu.CompilerParams(
            use_tc_tiling_on_sc=(wc % 128 == 0)),
        scratch_types=[
            pltpu.VMEM_SHARED((ACC_ROWS, wc), jnp.float32),
            pltpu.VMEM((ch_b, batch), jnp.int32),
            pltpu.VMEM((ch_b, batch), jnp.int32),
        ] + [pltpu.VMEM((batch, wc), jnp.float32) for _ in range(nslots)]
          + [pltpu.SemaphoreType.DMA for _ in range(2 * nslots)],
    )
    def spmm(t0, t1, srci, dsti, out0, out1, acc, srcb, dstb, *bufsems):
        bufs = bufsems[:nslots]
        gsem = bufsems[nslots:2 * nslots]
        ssem = bufsems[2 * nslots:]
        cid = lax.axis_index("c")
        sid = lax.axis_index("s")
        if feature_split:
            b0 = sid * nb
        else:
            b0 = cid * (nbt // 2) + sid * nb

        @pl.loop(0, batch)
        def _(r):
            for c in range(wc // 16):
                bufs[0][r, pl.ds(c * 16, 16)] = jnp.zeros((16,), jnp.float32)

        for k in range(ZERO_ROWS // batch):
            pltpu.sync_copy(bufs[0], acc.at[pl.ds(sid * ZERO_ROWS + k * batch,
                                                  batch)])
        plsc.subcore_barrier()

        def run(t_hbm):
            @pl.loop(0, n_chunks)
            def _(ch):
                pltpu.sync_copy(srci.at[pl.ds(b0 + ch * ch_b, ch_b)], srcb)
                pltpu.sync_copy(dsti.at[pl.ds(b0 + ch * ch_b, ch_b)], dstb)

                @pl.loop(0, ch_b, step=nslots)
                def _(j):
                    @pl.when(j > 0)
                    def _():
                        for k in range(nslots):
                            pltpu.make_async_copy(
                                bufs[k], acc.at[dstb.at[j - nslots + k]],
                                ssem[k]).wait()
                    cps = [pltpu.async_copy(t_hbm.at[srcb.at[j + k]], bufs[k],
                                            gsem[k]) for k in range(nslots)]
                    for k in range(nslots):
                        cps[k].wait()
                        pltpu.async_copy(bufs[k], acc.at[dstb.at[j + k]],
                                         ssem[k], add=True)

                for k in range(nslots):
                    pltpu.make_async_copy(
                        bufs[k], acc.at[dstb.at[ch_b - nslots + k]],
                        ssem[k]).wait()

        @pl.when(cid == 0)
        def _():
            run(t0)

        @pl.when(cid == 1)
        def _():
            run(t1)

        plsc.subcore_barrier()
        r0 = sid * OUT_PT

        def writeout(out):
            pltpu.sync_copy(acc.at[pl.ds(r0, OUT_PT)],
                            out.at[pl.ds(r0, OUT_PT)])

            @pl.when(sid == 15)
            def _():
                pltpu.sync_copy(acc.at[pl.ds(16 * OUT_PT, OUT_TAIL)],
                                out.at[pl.ds(16 * OUT_PT, OUT_TAIL)])

        @pl.when(cid == 0)
        def _():
            writeout(out0)

        @pl.when(cid == 1)
        def _():
            writeout(out1)

    return spmm


def _rows(cols):
    return pl.BlockSpec((BN, cols), lambda i: (i, 0))


def _full(shape):
    return pl.BlockSpec(shape, lambda i: tuple(0 for _ in shape))


def _dot(a, b):
    return jnp.dot(a, b, preferred_element_type=jnp.float32)


def _relu(v):
    return jnp.maximum(v, 0.0)


def _ae_body(x, e1w, e1b, e2w, e2b, e3w, e3b, zw, zb, d1w, d1b, d2w, d2b,
             d3w, d3b, xw, xb, cl, t1_o, t2_o, t3_o, z_o, xbar_o, q_o):
    xv = x[...]
    t1 = _relu(_dot(xv, e1w[...]) + e1b[...])
    t2 = _relu(_dot(t1, e2w[...]) + e2b[...])
    t3 = _relu(_dot(t2, e3w[...]) + e3b[...])
    z = _dot(t3, zw[...]) + zb[...]
    dh1 = _relu(_dot(z, d1w[...]) + d1b[...])
    dh2 = _relu(_dot(dh1, d2w[...]) + d2b[...])
    dh3 = _relu(_dot(dh2, d3w[...]) + d3b[...])
    xbar = _dot(dh3, xw[...]) + xb[...]
    c = cl[...]
    zc = lax.dot_general(z, c, (((1,), (1,)), ((), ())),
                         preferred_element_type=jnp.float32)
    sq = (jnp.sum(z * z, axis=1, keepdims=True)
          + jnp.sum(c * c, axis=1)[None, :] - 2.0 * zc)
    q = 1.0 / (1.0 + sq)
    q = q / jnp.sum(q, axis=1, keepdims=True)
    t1_o[...] = t1
    t2_o[...] = t2
    t3_o[...] = t3
    z_o[...] = z
    xbar_o[...] = xbar
    q_o[...] = q


def _ae_call(x, ws):
    f32 = jnp.float32
    return pl.pallas_call(
        _ae_body,
        grid=(GRID_R,),
        in_specs=[_rows(D)] + [_full(w.shape) for w in ws],
        out_specs=[_rows(256), _rows(256), _rows(256), _rows(NZ), _rows(D),
                   _rows(NC)],
        out_shape=[jax.ShapeDtypeStruct((N, 256), f32),
                   jax.ShapeDtypeStruct((N, 256), f32),
                   jax.ShapeDtypeStruct((N, 256), f32),
                   jax.ShapeDtypeStruct((N, NZ), f32),
                   jax.ShapeDtypeStruct((N, D), f32),
                   jax.ShapeDtypeStruct((N, NC), f32)],
    )(x, *ws)


def _mix1_body(p0, p1, g1, t1, ma_o, mb_o):
    h = _relu(_dot(p0[...] + p1[...], g1[...]))
    m = (1.0 - SIGMA) * h + SIGMA * t1[...]
    ma_o[...] = m[:, :D]
    mb_o[...] = m[:, D:]


def _mix1_call(p0, p1, g1, t1):
    f32 = jnp.float32
    return pl.pallas_call(
        _mix1_body,
        grid=(GRID_R,),
        in_specs=[_rows(D), _rows(D), _full(g1.shape), _rows(256)],
        out_specs=[_rows(D), _rows(D)],
        out_shape=[jax.ShapeDtypeStruct((N, D), f32),
                   jax.ShapeDtypeStruct((N, D), f32)],
    )(p0, p1, g1, t1)


def _mix2_body(aa, ab, g, t, ma_o, mb_o):
    gv = g[...]
    h = _relu(_dot(aa[...], gv[:D, :]) + _dot(ab[...], gv[D:, :]))
    m = (1.0 - SIGMA) * h + SIGMA * t[...]
    ma_o[...] = m[:, :D]
    mb_o[...] = m[:, D:]


def _mix2_call(aa, ab, g, t):
    f32 = jnp.float32
    return pl.pallas_call(
        _mix2_body,
        grid=(GRID_R,),
        in_specs=[_rows(D), _rows(D), _full(g.shape), _rows(256)],
        out_specs=[_rows(D), _rows(D)],
        out_shape=[jax.ShapeDtypeStruct((N, D), f32),
                   jax.ShapeDtypeStruct((N, D), f32)],
    )(aa, ab, g, t)


def _mix3_body(aa, ab, g3, t3, g4, s4_o):
    g3v = g3[...]
    h = _relu(_dot(aa[...], g3v[:D, :]) + _dot(ab[...], g3v[D:, :]))
    m = (1.0 - SIGMA) * h + SIGMA * t3[...]
    s4_o[...] = _dot(m, g4[...])


def _mix3_call(aa, ab, g3, t3, g4):
    return pl.pallas_call(
        _mix3_body,
        grid=(GRID_R,),
        in_specs=[_rows(D), _rows(D), _full(g3.shape), _rows(256),
                  _full(g4.shape)],
        out_specs=_rows(NZ),
        out_shape=jax.ShapeDtypeStruct((N, NZ), jnp.float32),
    )(aa, ab, g3, t3, g4)


def _mix4_body(p0, p1, z, g5, s5_o):
    h = _relu(p0[...] + p1[...])
    m = (1.0 - SIGMA) * h + SIGMA * z[...]
    s5_o[...] = _dot(m, g5[...])


def _mix4_call(p0, p1, z, g5):
    return pl.pallas_call(
        _mix4_body,
        grid=(GRID_R,),
        in_specs=[_rows(NZ), _rows(NZ), _rows(NZ), _full(g5.shape)],
        out_specs=_rows(NC),
        out_shape=jax.ShapeDtypeStruct((N, NC), jnp.float32),
    )(p0, p1, z, g5)


def _copy_body(a, o):
    o[...] = a[...]


def _copy_call(a):
    cols = a.shape[1]
    return pl.pallas_call(
        _copy_body,
        grid=(GRID_R,),
        in_specs=[_rows(cols)],
        out_specs=_rows(cols),
        out_shape=jax.ShapeDtypeStruct(a.shape, a.dtype),
    )(a)

def _fin_body(p0, p1, pred_o):
    h = p0[...] + p1[...]
    e = jnp.exp(h - jnp.max(h, axis=1, keepdims=True))
    pred_o[...] = e / jnp.sum(e, axis=1, keepdims=True)


def _fin_call(p0, p1):
    return pl.pallas_call(
        _fin_body,
        grid=(GRID_R,),
        in_specs=[_rows(NC), _rows(NC)],
        out_specs=_rows(NC),
        out_shape=jax.ShapeDtypeStruct((N, NC), jnp.float32),
    )(p0, p1)


def kernel(x, adj, e1w, e1b, e2w, e2b, e3w, e3b, zw, zb, d1w, d1b, d2w, d2b,
           d3w, d3b, xw, xb, g1, g2, g3, g4, g5, cluster):
    pad = NE_PAD - NE
    src = jnp.concatenate([adj[0], jnp.zeros((pad,), jnp.int32)])
    dst = jnp.concatenate([adj[1], jnp.full((pad,), N, jnp.int32)])
    src2d = src.reshape(NB_TOTAL, BATCH)
    dst2d = dst.reshape(NB_TOTAL, BATCH)

    r = lambda b: b.reshape(1, -1)
    ws = (e1w, r(e1b), e2w, r(e2b), e3w, r(e3b), zw, r(zb), d1w, r(d1b),
          d2w, r(d2b), d3w, r(d3b), xw, r(xb), cluster)
    t1, t2, t3, z, x_bar, q = _ae_call(x, ws)

    spmm128_es = _make_spmm(D, False, batch=64, nslots=4, n_chunks=2)
    spmm128_fs = _make_spmm(D, True, batch=64, nslots=4, n_chunks=4)
    spmm16_es = _make_spmm(NZ, False, batch=BATCH, nslots=4, n_chunks=1)
    src64 = src.reshape(NE_PAD // 64, 64)
    dst64 = dst.reshape(NE_PAD // 64, 64)

    x2 = _copy_call(x)
    p0, p1 = spmm128_es(x, x2, src64, dst64)
    m2a, m2b = _mix1_call(p0, p1, g1, t1)
    a2a, a2b = spmm128_fs(m2a, m2b, src64, dst64)
    m3a, m3b = _mix2_call(a2a, a2b, g2, t2)
    a3a, a3b = spmm128_fs(m3a, m3b, src64, dst64)
    s4 = _mix3_call(a3a, a3b, g3, t3, g4)
    p4a, p4b = spmm16_es(s4, _copy_call(s4), src2d, dst2d)
    s5 = _mix4_call(p4a, p4b, z, g5)
    p5a, p5b = spmm16_es(s5, _copy_call(s5), src2d, dst2d)
    predict = _fin_call(p5a, p5b)

    return (x_bar, q, predict, z)


# speedup vs baseline: 1.0639x; 1.0639x over previous
import functools

import jax
import jax.numpy as jnp
from jax import lax
from jax.experimental import pallas as pl
from jax.experimental.pallas import tpu as pltpu
from jax.experimental.pallas import tpu_sc as plsc

N = 10000
D = 128
NZ = 16
NC = 16
NE = 160000
SIGMA = 0.5

BATCH = 128
NE_PAD = 163840
NB_TOTAL = NE_PAD // BATCH
ACC_ROWS = 10240
ZERO_ROWS = ACC_ROWS // 16
OUT_PT = 624
OUT_TAIL = N - 16 * OUT_PT

BN = 1000
GRID_R = N // BN


@functools.lru_cache(maxsize=None)
def _make_spmm(wc: int, feature_split: bool, batch: int = BATCH,
               nslots: int = 2, n_chunks: int = 5):
    nbt = NE_PAD // batch
    nb = (nbt // 16) if feature_split else (nbt // 32)
    ch_b = nb // n_chunks
    assert ch_b % 8 == 0 and ch_b % nslots == 0, (ch_b, nslots)
    mesh = plsc.VectorSubcoreMesh(core_axis_name="c", subcore_axis_name="s")
    out_sds = jax.ShapeDtypeStruct((N, wc), jnp.float32)

    @functools.partial(
        pl.kernel,
        out_type=(out_sds, out_sds),
        mesh=mesh,
        compiler_params=pltpu.CompilerParams(
            use_tc_tiling_on_sc=(wc % 128 == 0)),
        scratch_types=[
            pltpu.VMEM_SHARED((ACC_ROWS, wc), jnp.float32),
            pltpu.VMEM((ch_b, batch), jnp.int32),
            pltpu.VMEM((ch_b, batch), jnp.int32),
        ] + [pltpu.VMEM((batch, wc), jnp.float32) for _ in range(nslots)]
          + [pltpu.SemaphoreType.DMA for _ in range(2 * nslots)],
    )
    def spmm(t0, t1, srci, dsti, out0, out1, acc, srcb, dstb, *bufsems):
        bufs = bufsems[:nslots]
        gsem = bufsems[nslots:2 * nslots]
        ssem = bufsems[2 * nslots:]
        cid = lax.axis_index("c")
        sid = lax.axis_index("s")
        if feature_split:
            b0 = sid * nb
        else:
            b0 = cid * (nbt // 2) + sid * nb

        @pl.loop(0, batch)
        def _(r):
            for c in range(wc // 16):
                bufs[0][r, pl.ds(c * 16, 16)] = jnp.zeros((16,), jnp.float32)

        for k in range(ZERO_ROWS // batch):
            pltpu.sync_copy(bufs[0], acc.at[pl.ds(sid * ZERO_ROWS + k * batch,
                                                  batch)])
        plsc.subcore_barrier()

        def run(t_hbm):
            @pl.loop(0, n_chunks)
            def _(ch):
                pltpu.sync_copy(srci.at[pl.ds(b0 + ch * ch_b, ch_b)], srcb)
                pltpu.sync_copy(dsti.at[pl.ds(b0 + ch * ch_b, ch_b)], dstb)

                @pl.loop(0, ch_b, step=nslots)
                def _(j):
                    @pl.when(j > 0)
                    def _():
                        for k in range(nslots):
                            pltpu.make_async_copy(
                                bufs[k], acc.at[dstb.at[j - nslots + k]],
                                ssem[k]).wait()
                    cps = [pltpu.async_copy(t_hbm.at[srcb.at[j + k]], bufs[k],
                                            gsem[k]) for k in range(nslots)]
                    for k in range(nslots):
                        cps[k].wait()
                        pltpu.async_copy(bufs[k], acc.at[dstb.at[j + k]],
                                         ssem[k], add=True)

                for k in range(nslots):
                    pltpu.make_async_copy(
                        bufs[k], acc.at[dstb.at[ch_b - nslots + k]],
                        ssem[k]).wait()

        @pl.when(cid == 0)
        def _():
            run(t0)

        @pl.when(cid == 1)
        def _():
            run(t1)

        plsc.subcore_barrier()
        r0 = sid * OUT_PT

        def writeout(out):
            pltpu.sync_copy(acc.at[pl.ds(r0, OUT_PT)],
                            out.at[pl.ds(r0, OUT_PT)])

            @pl.when(sid == 15)
            def _():
                pltpu.sync_copy(acc.at[pl.ds(16 * OUT_PT, OUT_TAIL)],
                                out.at[pl.ds(16 * OUT_PT, OUT_TAIL)])

        @pl.when(cid == 0)
        def _():
            writeout(out0)

        @pl.when(cid == 1)
        def _():
            writeout(out1)

    return spmm


def _rows(cols):
    return pl.BlockSpec((BN, cols), lambda i: (i, 0))


def _full(shape):
    return pl.BlockSpec(shape, lambda i: tuple(0 for _ in shape))


def _dot(a, b):
    return jnp.dot(a, b, preferred_element_type=jnp.float32)


def _relu(v):
    return jnp.maximum(v, 0.0)


def _ae_body(x, e1w, e1b, e2w, e2b, e3w, e3b, zw, zb, d1w, d1b, d2w, d2b,
             d3w, d3b, xw, xb, cl, t1_o, t2_o, t3_o, z_o, xbar_o, q_o):
    xv = x[...]
    t1 = _relu(_dot(xv, e1w[...]) + e1b[...])
    t2 = _relu(_dot(t1, e2w[...]) + e2b[...])
    t3 = _relu(_dot(t2, e3w[...]) + e3b[...])
    z = _dot(t3, zw[...]) + zb[...]
    dh1 = _relu(_dot(z, d1w[...]) + d1b[...])
    dh2 = _relu(_dot(dh1, d2w[...]) + d2b[...])
    dh3 = _relu(_dot(dh2, d3w[...]) + d3b[...])
    xbar = _dot(dh3, xw[...]) + xb[...]
    c = cl[...]
    zc = lax.dot_general(z, c, (((1,), (1,)), ((), ())),
                         preferred_element_type=jnp.float32)
    sq = (jnp.sum(z * z, axis=1, keepdims=True)
          + jnp.sum(c * c, axis=1)[None, :] - 2.0 * zc)
    q = 1.0 / (1.0 + sq)
    q = q / jnp.sum(q, axis=1, keepdims=True)
    t1_o[...] = t1
    t2_o[...] = t2
    t3_o[...] = t3
    z_o[...] = z
    xbar_o[...] = xbar
    q_o[...] = q


def _ae_call(x, ws):
    f32 = jnp.float32
    return pl.pallas_call(
        _ae_body,
        grid=(GRID_R,),
        in_specs=[_rows(D)] + [_full(w.shape) for w in ws],
        out_specs=[_rows(256), _rows(256), _rows(256), _rows(NZ), _rows(D),
                   _rows(NC)],
        out_shape=[jax.ShapeDtypeStruct((N, 256), f32),
                   jax.ShapeDtypeStruct((N, 256), f32),
                   jax.ShapeDtypeStruct((N, 256), f32),
                   jax.ShapeDtypeStruct((N, NZ), f32),
                   jax.ShapeDtypeStruct((N, D), f32),
                   jax.ShapeDtypeStruct((N, NC), f32)],
    )(x, *ws)


def _mix1_body(p0, p1, g1, t1, ma_o, mb_o):
    h = _relu(_dot(p0[...] + p1[...], g1[...]))
    m = (1.0 - SIGMA) * h + SIGMA * t1[...]
    ma_o[...] = m[:, :D]
    mb_o[...] = m[:, D:]


def _mix1_call(p0, p1, g1, t1):
    f32 = jnp.float32
    return pl.pallas_call(
        _mix1_body,
        grid=(GRID_R,),
        in_specs=[_rows(D), _rows(D), _full(g1.shape), _rows(256)],
        out_specs=[_rows(D), _rows(D)],
        out_shape=[jax.ShapeDtypeStruct((N, D), f32),
                   jax.ShapeDtypeStruct((N, D), f32)],
    )(p0, p1, g1, t1)


def _mix2_body(aa, ab, g, t, ma_o, mb_o):
    gv = g[...]
    h = _relu(_dot(aa[...], gv[:D, :]) + _dot(ab[...], gv[D:, :]))
    m = (1.0 - SIGMA) * h + SIGMA * t[...]
    ma_o[...] = m[:, :D]
    mb_o[...] = m[:, D:]


def _mix2_call(aa, ab, g, t):
    f32 = jnp.float32
    return pl.pallas_call(
        _mix2_body,
        grid=(GRID_R,),
        in_specs=[_rows(D), _rows(D), _full(g.shape), _rows(256)],
        out_specs=[_rows(D), _rows(D)],
        out_shape=[jax.ShapeDtypeStruct((N, D), f32),
                   jax.ShapeDtypeStruct((N, D), f32)],
    )(aa, ab, g, t)


def _mix3_body(aa, ab, g3, t3, g4, s4_o):
    g3v = g3[...]
    h = _relu(_dot(aa[...], g3v[:D, :]) + _dot(ab[...], g3v[D:, :]))
    m = (1.0 - SIGMA) * h + SIGMA * t3[...]
    s4_o[...] = _dot(m, g4[...])


def _mix3_call(aa, ab, g3, t3, g4):
    return pl.pallas_call(
        _mix3_body,
        grid=(GRID_R,),
        in_specs=[_rows(D), _rows(D), _full(g3.shape), _rows(256),
                  _full(g4.shape)],
        out_specs=_rows(NZ),
        out_shape=jax.ShapeDtypeStruct((N, NZ), jnp.float32),
    )(aa, ab, g3, t3, g4)


def _mix4_body(p0, p1, z, g5, s5_o):
    h = _relu(p0[...] + p1[...])
    m = (1.0 - SIGMA) * h + SIGMA * z[...]
    s5_o[...] = _dot(m, g5[...])


def _mix4_call(p0, p1, z, g5):
    return pl.pallas_call(
        _mix4_body,
        grid=(GRID_R,),
        in_specs=[_rows(NZ), _rows(NZ), _rows(NZ), _full(g5.shape)],
        out_specs=_rows(NC),
        out_shape=jax.ShapeDtypeStruct((N, NC), jnp.float32),
    )(p0, p1, z, g5)


def _fin_body(p0, p1, pred_o):
    h = p0[...] + p1[...]
    e = jnp.exp(h - jnp.max(h, axis=1, keepdims=True))
    pred_o[...] = e / jnp.sum(e, axis=1, keepdims=True)


def _fin_call(p0, p1):
    return pl.pallas_call(
        _fin_body,
        grid=(GRID_R,),
        in_specs=[_rows(NC), _rows(NC)],
        out_specs=_rows(NC),
        out_shape=jax.ShapeDtypeStruct((N, NC), jnp.float32),
    )(p0, p1)


def kernel(x, adj, e1w, e1b, e2w, e2b, e3w, e3b, zw, zb, d1w, d1b, d2w, d2b,
           d3w, d3b, xw, xb, g1, g2, g3, g4, g5, cluster):
    pad = NE_PAD - NE
    src = jnp.concatenate([adj[0], jnp.zeros((pad,), jnp.int32)])
    dst = jnp.concatenate([adj[1], jnp.full((pad,), N, jnp.int32)])
    src2d = src.reshape(NB_TOTAL, BATCH)
    dst2d = dst.reshape(NB_TOTAL, BATCH)

    r = lambda b: b.reshape(1, -1)
    ws = (e1w, r(e1b), e2w, r(e2b), e3w, r(e3b), zw, r(zb), d1w, r(d1b),
          d2w, r(d2b), d3w, r(d3b), xw, r(xb), cluster)
    t1, t2, t3, z, x_bar, q = _ae_call(x, ws)

    spmm128_es = _make_spmm(D, False, batch=64, nslots=4, n_chunks=2)
    spmm128_fs = _make_spmm(D, True, batch=64, nslots=4, n_chunks=4)
    spmm16_es = _make_spmm(NZ, False, batch=BATCH, nslots=4, n_chunks=1)
    src64 = src.reshape(NE_PAD // 64, 64)
    dst64 = dst.reshape(NE_PAD // 64, 64)

    p0, p1 = spmm128_es(x, x, src64, dst64)
    m2a, m2b = _mix1_call(p0, p1, g1, t1)
    a2a, a2b = spmm128_fs(m2a, m2b, src64, dst64)
    m3a, m3b = _mix2_call(a2a, a2b, g2, t2)
    a3a, a3b = spmm128_fs(m3a, m3b, src64, dst64)
    s4 = _mix3_call(a3a, a3b, g3, t3, g4)
    p4a, p4b = spmm16_es(s4, s4, src2d, dst2d)
    s5 = _mix4_call(p4a, p4b, z, g5)
    p5a, p5b = spmm16_es(s5, s5, src2d, dst2d)
    predict = _fin_call(p5a, p5b)

    return (x_bar, q, predict, z)
